# baseline (speedup 1.0000x reference)
import functools

import jax
import jax.numpy as jnp
from jax import lax
from jax.experimental import pallas as pl
from jax.experimental.pallas import tpu as pltpu
from jax.experimental.pallas import tpu_sc as plsc

_NC = 2
_NS = 16
_NW = _NC * _NS
_NB = 4


def _emb(table, words):
    V, D = table.shape
    B, L = words.shape
    RPT = B // _NW
    n_grp = RPT // _NB

    mesh = plsc.VectorSubcoreMesh(core_axis_name="c", subcore_axis_name="s")

    @functools.partial(
        pl.kernel,
        mesh=mesh,
        out_type=jax.ShapeDtypeStruct((B, L, D), jnp.float32),
        scratch_types=[
            pltpu.VMEM((RPT, L), jnp.int32),
            pltpu.VMEM((_NB, L, D), jnp.float32),
            pltpu.SemaphoreType.DMA,
            pltpu.SemaphoreType.DMA,
        ],
        compiler_params=pltpu.CompilerParams(use_tc_tiling_on_sc=False),
    )
    def emb_gather(table_hbm, words_hbm, out_hbm, idx_v, bufs, gsem, osem):
        cid = lax.axis_index("c")
        sid = lax.axis_index("s")
        wid = sid * _NC + cid
        w0 = wid * RPT
        pltpu.sync_copy(words_hbm.at[pl.ds(w0, RPT)], idx_v)

        def fire(r, slot):
            pltpu.async_copy(
                table_hbm.at[idx_v.at[r]], bufs.at[slot], gsem)

        def wait_gather(slot):
            pltpu.make_async_copy(
                table_hbm.at[idx_v.at[0]], bufs.at[slot], gsem).wait()

        def wait_store():
            pltpu.make_async_copy(bufs.at[0], out_hbm.at[w0], osem).wait()

        fire(0, 0)

        def body(g, carry):
            for j in range(_NB):
                r = g * _NB + j
                if j == _NB - 1:
                    wait_store()
                else:
                    @pl.when(g > 0)
                    def _():
                        wait_store()
                if j == _NB - 1:
                    @pl.when(g < n_grp - 1)
                    def _():
                        fire(r + 1, 0)
                else:
                    fire(r + 1, j + 1)
                wait_gather(j)
                pltpu.async_copy(bufs.at[j], out_hbm.at[w0 + r], osem)
            return carry

        lax.fori_loop(0, n_grp, body, 0)
        wait_store()
        wait_store()
        wait_store()

    return emb_gather(table, words)


def kernel(table, words):
    L = words.shape[1]
    C0 = 128
    out_a = _emb(table, words[:, :C0])
    out_b = _emb(table, words[:, C0:])
    return jnp.concatenate([out_a, out_b], axis=1)


# speedup vs baseline: 1.2028x; 1.2028x over previous
import functools

import jax
import jax.numpy as jnp
from jax import lax
from jax.experimental import pallas as pl
from jax.experimental.pallas import tpu as pltpu
from jax.experimental.pallas import tpu_sc as plsc

_NC = 2
_NS = 16
_NW = _NC * _NS
_NB = 4


def _emb(table, words):
    V, D = table.shape
    B, L = words.shape
    RPT = B // _NW
    C0 = 128
    C1 = L - C0
    n_grp = RPT // _NB

    mesh = plsc.VectorSubcoreMesh(core_axis_name="c", subcore_axis_name="s")

    @functools.partial(
        pl.kernel,
        mesh=mesh,
        out_type=jax.ShapeDtypeStruct((B, L, D), jnp.float32),
        scratch_types=[
            pltpu.VMEM((RPT, L), jnp.int32),
            pltpu.VMEM((_NB, L, D), jnp.float32),
            pltpu.SemaphoreType.DMA,
            pltpu.SemaphoreType.DMA,
        ],
        compiler_params=pltpu.CompilerParams(use_tc_tiling_on_sc=False),
    )
    def emb_gather(table_hbm, words_hbm, out_hbm, idx_v, bufs, gsem, osem):
        cid = lax.axis_index("c")
        sid = lax.axis_index("s")
        wid = sid * _NC + cid
        w0 = wid * RPT
        pltpu.sync_copy(words_hbm.at[pl.ds(w0, RPT)], idx_v)

        def fire(r, slot):
            pltpu.async_copy(
                table_hbm.at[idx_v.at[r, pl.ds(0, C0)]],
                bufs.at[slot, pl.ds(0, C0)], gsem)
            pltpu.async_copy(
                table_hbm.at[idx_v.at[r, pl.ds(C0, C1)]],
                bufs.at[slot, pl.ds(C0, C1)], gsem)

        def wait_gather(slot):
            pltpu.make_async_copy(
                table_hbm.at[idx_v.at[0]], bufs.at[slot], gsem).wait()

        def wait_store():
            pltpu.make_async_copy(bufs.at[0], out_hbm.at[w0], osem).wait()

        fire(0, 0)

        def body(g, carry):
            for j in range(_NB):
                r = g * _NB + j
                if j == _NB - 1:
                    wait_store()
                else:
                    @pl.when(g > 0)
                    def _():
                        wait_store()
                if j == _NB - 1:
                    @pl.when(g < n_grp - 1)
                    def _():
                        fire(r + 1, 0)
                else:
                    fire(r + 1, j + 1)
                wait_gather(j)
                pltpu.async_copy(bufs.at[j], out_hbm.at[w0 + r], osem)
            return carry

        lax.fori_loop(0, n_grp, body, 0)
        wait_store()
        wait_store()
        wait_store()

    return emb_gather(table, words)


def kernel(table, words):
    return _emb(table, words)
